# baseline (speedup 1.0000x reference)
import functools

import jax
import jax.numpy as jnp
from jax import lax
from jax.experimental import pallas as pl
from jax.experimental.pallas import tpu as pltpu
from jax.experimental.pallas import tpu_sc as plsc


def _stage_a(x, A):
    bs, N, C = x.shape
    S, _, O = A.shape
    TN = 2000
    assert N % TN == 0

    def body(x_ref, a_ref, y_ref):
        for b in range(bs):
            y_ref[0, :, b * O:(b + 1) * O] = jnp.dot(
                x_ref[b], a_ref[0], preferred_element_type=jnp.float32)

    return pl.pallas_call(
        body,
        grid=(N // TN, S),
        in_specs=[
            pl.BlockSpec((bs, TN, C), lambda nt, s: (0, nt, 0)),
            pl.BlockSpec((1, C, O), lambda nt, s: (s, 0, 0)),
        ],
        out_specs=pl.BlockSpec((1, TN, bs * O), lambda nt, s: (s, nt, 0)),
        out_shape=jax.ShapeDtypeStruct((S, N, bs * O), jnp.float32),
    )(x, A)


def _stage_b(offs, yflat, bs, S, NH, O, CB):
    info = plsc.get_sparse_core_info()
    NC, NS = info.num_cores, info.num_subcores
    NW = NC * NS
    BO = bs * O
    npw = NH // NW
    nblk = npw // CB
    nofs = S * npw
    assert npw * NW == NH and nblk * CB == npw and CB % 8 == 0

    mesh = plsc.VectorSubcoreMesh(core_axis_name="c", subcore_axis_name="s")

    @functools.partial(
        pl.kernel,
        out_type=jax.ShapeDtypeStruct((bs, NH, O), jnp.float32),
        mesh=mesh,
        scratch_types=[
            pltpu.VMEM((nofs,), jnp.int32),
            pltpu.VMEM((S, CB, BO), jnp.float32),
            pltpu.VMEM((bs, CB, O), jnp.float32),
            pltpu.SemaphoreType.DMA,
        ],
    )
    def k(offs_hbm, y_hbm, out_hbm, offs_v, rows_v, h_v, sem):
        cid = lax.axis_index("c")
        sid = lax.axis_index("s")
        wid = sid * NC + cid
        base = wid * npw
        pltpu.sync_copy(offs_hbm.at[pl.ds(wid * nofs, nofs)], offs_v)

        NSPLIT = 2
        H = CB // NSPLIT

        def one_chunk(j):
            nb = base + j * CB
            cps = [pltpu.async_copy(
                       y_hbm.at[offs_v.at[pl.ds(s * npw + j * CB + t * H, H)]],
                       rows_v.at[s, pl.ds(t * H, H)], sem)
                   for s in range(S) for t in range(NSPLIT)]
            for cp in cps:
                cp.wait()

            @plsc.parallel_loop(0, CB)
            def comb(i):
                for bb in range(bs):
                    for c in range(O // 16):
                        sl = pl.ds(bb * O + c * 16, 16)
                        vs = [rows_v[s, i, sl] for s in range(S)]
                        while len(vs) > 1:
                            vs = [vs[k2] + vs[k2 + 1]
                                  for k2 in range(0, len(vs) - 1, 2)] \
                                 + ([vs[-1]] if len(vs) % 2 else [])
                        h_v[bb, i, pl.ds(c * 16, 16)] = vs[0]

            for bb in range(bs):
                pltpu.sync_copy(h_v.at[bb], out_hbm.at[bb, pl.ds(nb, CB)])

        lax.fori_loop(0, nblk, lambda j, c: (one_chunk(j), c)[1], 0)

    return k(offs, yflat)


def _stage_c(dt, h, bias2d, prev, bs, M, N, O, BK, NH, koff):
    nk = NH // BK
    assert nk * BK == NH

    def body(dt_ref, h_ref, b_ref, *rest):
        if prev is None:
            out_ref, = rest
        else:
            p_ref, out_ref = rest
        kk = pl.program_id(0)

        @pl.when(kk == 0)
        def _():
            if prev is None:
                out_ref[...] = jnp.zeros_like(out_ref)
            else:
                out_ref[...] = p_ref[...]

        rem = N - (koff + kk) * BK
        col = lax.broadcasted_iota(jnp.int32, (1, BK), 1)
        dtb = jnp.where(col < rem, dt_ref[...], 0.0)
        hb = h_ref[...] + b_ref[...][None]
        eh = jnp.where(hb > 0, hb, jnp.exp(jnp.minimum(hb, 0.0)) - 1.0)
        row = lax.broadcasted_iota(jnp.int32, (1, BK, 1), 1)
        eh = jnp.where(row < rem, eh, 0.0)
        for b in range(bs):
            out_ref[b] += jnp.dot(dtb, eh[b], preferred_element_type=jnp.float32)

    in_specs = [
        pl.BlockSpec((M, BK), lambda k: (0, k)),
        pl.BlockSpec((bs, BK, O), lambda k: (0, k, 0)),
        pl.BlockSpec((1, O), lambda k: (0, 0)),
    ]
    args = [dt, h, bias2d]
    if prev is not None:
        in_specs.append(pl.BlockSpec((bs, M, O), lambda k: (0, 0, 0)))
        args.append(prev)

    return pl.pallas_call(
        body,
        grid=(nk,),
        in_specs=in_specs,
        out_specs=pl.BlockSpec((bs, M, O), lambda k: (0, 0, 0)),
        out_shape=jax.ShapeDtypeStruct((bs, M, O), jnp.float32),
    )(*args)


def kernel(x, down_transform, indices, W, b):
    bs, N, C = x.shape
    _, S = indices.shape
    O = W.shape[0]
    M = down_transform.shape[0]

    CB = 16
    NW = 32
    BK = 512
    chunk = NW * CB
    NPAD = ((N + 2 * chunk - 1) // (2 * chunk)) * (2 * chunk)
    NH = NPAD // 2
    npw = NH // NW
    assert NH % BK == 0

    A = jnp.transpose(W.reshape(O, S, C), (1, 2, 0))
    Y = _stage_a(x, A)
    yflat = Y.reshape(S * N, bs * O)

    idx_pad = jnp.pad(indices, ((0, NPAD - N), (0, 0)))
    offs = idx_pad.T + (jnp.arange(S, dtype=jnp.int32) * N)[:, None]
    offs = offs.reshape(S, 2, NW, npw).transpose(1, 2, 0, 3).reshape(2, -1)

    bias2d = b.reshape(1, O)
    dtp = jnp.pad(down_transform, ((0, 0), (0, NPAD - N)))

    h1 = _stage_b(offs[0], yflat, bs, S, NH, O, CB)
    h2 = _stage_b(offs[1], yflat, bs, S, NH, O, CB)
    out = _stage_c(dtp[:, :NH], h1, bias2d, None, bs, M, N, O, BK, NH, 0)
    out = _stage_c(dtp[:, NH:], h2, bias2d, out, bs, M, N, O, BK, NH, NH // BK)
    return out


# speedup vs baseline: 1.0736x; 1.0736x over previous
import functools

import jax
import jax.numpy as jnp
from jax import lax
from jax.experimental import pallas as pl
from jax.experimental.pallas import tpu as pltpu
from jax.experimental.pallas import tpu_sc as plsc


def _stage_a(x, A):
    bs, N, C = x.shape
    S, _, O = A.shape
    TN = 2000
    assert N % TN == 0

    def body(x_ref, a_ref, y_ref):
        for b in range(bs):
            y_ref[0, :, b * O:(b + 1) * O] = jnp.dot(
                x_ref[b], a_ref[0], preferred_element_type=jnp.float32)

    return pl.pallas_call(
        body,
        grid=(N // TN, S),
        in_specs=[
            pl.BlockSpec((bs, TN, C), lambda nt, s: (0, nt, 0)),
            pl.BlockSpec((1, C, O), lambda nt, s: (s, 0, 0)),
        ],
        out_specs=pl.BlockSpec((1, TN, bs * O), lambda nt, s: (s, nt, 0)),
        out_shape=jax.ShapeDtypeStruct((S, N, bs * O), jnp.float32),
    )(x, A)


def _stage_b(offs, yflat, bs, S, NPAD, O, CB):
    info = plsc.get_sparse_core_info()
    NC, NS = info.num_cores, info.num_subcores
    NW = NC * NS
    BO = bs * O
    npw = NPAD // NW
    nblk = npw // CB
    nofs = S * npw
    assert npw * NW == NPAD and nblk * CB == npw and CB % 8 == 0

    mesh = plsc.VectorSubcoreMesh(core_axis_name="c", subcore_axis_name="s")

    @functools.partial(
        pl.kernel,
        out_type=jax.ShapeDtypeStruct((bs, NPAD, O), jnp.float32),
        mesh=mesh,
        scratch_types=[
            pltpu.VMEM((nofs,), jnp.int32),
            pltpu.VMEM((S, CB, BO), jnp.float32),
            pltpu.VMEM((bs, CB, O), jnp.float32),
            pltpu.SemaphoreType.DMA,
        ],
    )
    def k(offs_hbm, y_hbm, out_hbm, offs_v, rows_v, h_v, sem):
        cid = lax.axis_index("c")
        sid = lax.axis_index("s")
        wid = sid * NC + cid
        base = wid * npw
        pltpu.sync_copy(offs_hbm.at[pl.ds(wid * nofs, nofs)], offs_v)

        NSPLIT = 2
        H = CB // NSPLIT

        def one_chunk(j):
            nb = base + j * CB
            cps = [pltpu.async_copy(
                       y_hbm.at[offs_v.at[pl.ds(s * npw + j * CB + t * H, H)]],
                       rows_v.at[s, pl.ds(t * H, H)], sem)
                   for s in range(S) for t in range(NSPLIT)]
            for cp in cps:
                cp.wait()

            @plsc.parallel_loop(0, CB)
            def comb(i):
                for bb in range(bs):
                    for c in range(O // 16):
                        sl = pl.ds(bb * O + c * 16, 16)
                        vs = [rows_v[s, i, sl] for s in range(S)]
                        while len(vs) > 1:
                            vs = [vs[k2] + vs[k2 + 1]
                                  for k2 in range(0, len(vs) - 1, 2)] \
                                 + ([vs[-1]] if len(vs) % 2 else [])
                        h_v[bb, i, pl.ds(c * 16, 16)] = vs[0]

            for bb in range(bs):
                pltpu.sync_copy(h_v.at[bb], out_hbm.at[bb, pl.ds(nb, CB)])

        lax.fori_loop(0, nblk, lambda j, c: (one_chunk(j), c)[1], 0)

    return k(offs, yflat)


def _stage_c(dt, h, bias2d, bs, M, N, O, BK, NPAD):
    nk = NPAD // BK
    assert nk * BK == NPAD

    def body(dt_ref, h_ref, b_ref, out_ref):
        kk = pl.program_id(0)

        @pl.when(kk == 0)
        def _():
            out_ref[...] = jnp.zeros_like(out_ref)

        rem = N - kk * BK
        col = lax.broadcasted_iota(jnp.int32, (1, BK), 1)
        dtb = jnp.where(col < rem, dt_ref[...], 0.0)
        hb = h_ref[...] + b_ref[...][None]
        eh = jnp.where(hb > 0, hb, jnp.exp(jnp.minimum(hb, 0.0)) - 1.0)
        row = lax.broadcasted_iota(jnp.int32, (1, BK, 1), 1)
        eh = jnp.where(row < rem, eh, 0.0)
        for b in range(bs):
            out_ref[b] += jnp.dot(dtb, eh[b], preferred_element_type=jnp.float32)

    return pl.pallas_call(
        body,
        grid=(nk,),
        in_specs=[
            pl.BlockSpec((M, BK), lambda k: (0, k)),
            pl.BlockSpec((bs, BK, O), lambda k: (0, k, 0)),
            pl.BlockSpec((1, O), lambda k: (0, 0)),
        ],
        out_specs=pl.BlockSpec((bs, M, O), lambda k: (0, 0, 0)),
        out_shape=jax.ShapeDtypeStruct((bs, M, O), jnp.float32),
    )(dt, h, bias2d)


def kernel(x, down_transform, indices, W, b):
    bs, N, C = x.shape
    _, S = indices.shape
    O = W.shape[0]
    M = down_transform.shape[0]

    CB = 16
    NW = 32
    chunk = NW * CB
    NPAD = ((N + chunk - 1) // chunk) * chunk
    npw = NPAD // NW

    A = jnp.transpose(W.reshape(O, S, C), (1, 2, 0))
    Y = _stage_a(x, A)
    yflat = Y.reshape(S * N, bs * O)

    idx_pad = jnp.pad(indices, ((0, NPAD - N), (0, 0)))
    offs = idx_pad.T + (jnp.arange(S, dtype=jnp.int32) * N)[:, None]
    offs = offs.reshape(S, NW, npw).transpose(1, 0, 2).reshape(-1)

    h = _stage_b(offs, yflat, bs, S, NPAD, O, CB)
    out = _stage_c(down_transform, h, b.reshape(1, O), bs, M, N, O, 512, NPAD)
    return out
